# Initial kernel scaffold; baseline (speedup 1.0000x reference)
import jax
import jax.numpy as jnp
from jax.experimental import pallas as pl


def kernel(x, w_qkv_t, w_proj_t, b_qkv, b_proj):
    raise NotImplementedError("write your pallas kernel here")


# speedup vs baseline: 3.8744x; 3.8744x over previous
import functools

import jax
import jax.numpy as jnp
from jax import lax
from jax.experimental import pallas as pl
from jax.experimental.pallas import tpu as pltpu


def _fused_mha_kernel(x_ref, wqkv_ref, bqkv_ref, wproj_ref, bproj_ref,
                      o_ref, *, num_heads, head_dim, scale):
    D = num_heads * head_dim
    xb = x_ref[0]

    qkv = jnp.dot(xb, wqkv_ref[...],
                  preferred_element_type=jnp.float32)
    qkv = qkv + bqkv_ref[...]

    qb = (qkv[:, 0:D] * scale).astype(jnp.bfloat16)
    kb = qkv[:, D:2 * D].astype(jnp.bfloat16)
    vb = qkv[:, 2 * D:3 * D].astype(jnp.bfloat16)

    outs = []
    for h in range(num_heads):
        lo = h * head_dim
        hi = lo + head_dim
        s = lax.dot_general(qb[:, lo:hi], kb[:, lo:hi],
                            (((1,), (1,)), ((), ())),
                            preferred_element_type=jnp.float32)
        s = s - jnp.max(s, axis=-1, keepdims=True)
        p = jnp.exp(s)
        l = jnp.sum(p, axis=-1, keepdims=True)
        o = jnp.dot(p.astype(jnp.bfloat16), vb[:, lo:hi],
                    preferred_element_type=jnp.float32)
        outs.append(o * (1.0 / l))

    y = jnp.concatenate(outs, axis=-1).astype(jnp.bfloat16)
    out = jnp.dot(y, wproj_ref[...],
                  preferred_element_type=jnp.float32) + bproj_ref[...]
    o_ref[0] = out.astype(o_ref.dtype)


def kernel(x, w_qkv_t, w_proj_t, b_qkv, b_proj):
    B, N, D = x.shape
    num_heads = 12
    head_dim = D // num_heads
    scale = head_dim ** (-0.5)

    xb = x.astype(jnp.bfloat16)
    wq = w_qkv_t.astype(jnp.bfloat16)
    wp = w_proj_t.astype(jnp.bfloat16)
    bq = b_qkv.reshape(1, 3 * D)
    bp = b_proj.reshape(1, D)

    kern = functools.partial(_fused_mha_kernel, num_heads=num_heads,
                             head_dim=head_dim, scale=scale)
    return pl.pallas_call(
        kern,
        out_shape=jax.ShapeDtypeStruct((B, N, D), x.dtype),
        grid=(B,),
        in_specs=[
            pl.BlockSpec((1, N, D), lambda b: (b, 0, 0)),
            pl.BlockSpec((D, 3 * D), lambda b: (0, 0)),
            pl.BlockSpec((1, 3 * D), lambda b: (0, 0)),
            pl.BlockSpec((D, D), lambda b: (0, 0)),
            pl.BlockSpec((1, D), lambda b: (0, 0)),
        ],
        out_specs=pl.BlockSpec((1, N, D), lambda b: (b, 0, 0)),
        compiler_params=pltpu.CompilerParams(
            dimension_semantics=("parallel",),
            vmem_limit_bytes=100 * 1024 * 1024),
    )(xb, wq, bq, wp, bp)


# speedup vs baseline: 5.3405x; 1.3784x over previous
import functools

import jax
import jax.numpy as jnp
from jax import lax
from jax.experimental import pallas as pl
from jax.experimental.pallas import tpu as pltpu


def _fused_mha_kernel(x_ref, wqkv_ref, bqkv_ref, wproj_ref, bproj_ref,
                      o_ref, *, num_heads, head_dim, scale, bb, n):
    D = num_heads * head_dim
    xb = x_ref[...].reshape(bb * n, D).astype(jnp.bfloat16)

    qkv = jnp.dot(xb, wqkv_ref[...],
                  preferred_element_type=jnp.float32)
    qkv = qkv + bqkv_ref[...]

    qb = (qkv[:, 0:D] * scale).astype(jnp.bfloat16)
    kb = qkv[:, D:2 * D].astype(jnp.bfloat16)
    vb = qkv[:, 2 * D:3 * D].astype(jnp.bfloat16)

    outs = []
    for b in range(bb):
        r0 = b * n
        for h in range(num_heads):
            lo = h * head_dim
            hi = lo + head_dim
            qh = qb[r0:r0 + n, lo:hi]
            kh = kb[r0:r0 + n, lo:hi]
            vh = vb[r0:r0 + n, lo:hi]
            s = lax.dot_general(qh, kh, (((1,), (1,)), ((), ())),
                                preferred_element_type=jnp.float32)
            s = s - jnp.max(s, axis=-1, keepdims=True)
            p = jnp.exp(s)
            l = jnp.sum(p, axis=-1, keepdims=True)
            o = jnp.dot(p.astype(jnp.bfloat16), vh,
                        preferred_element_type=jnp.float32)
            outs.append(o * (1.0 / l))

    y = jnp.concatenate(
        [jnp.concatenate(outs[b * num_heads:(b + 1) * num_heads], axis=-1)
         for b in range(bb)], axis=0).astype(jnp.bfloat16)
    out = jnp.dot(y, wproj_ref[...],
                  preferred_element_type=jnp.float32) + bproj_ref[...]
    o_ref[...] = out.reshape(bb, n, D).astype(o_ref.dtype)


def kernel(x, w_qkv_t, w_proj_t, b_qkv, b_proj):
    B, N, D = x.shape
    num_heads = 12
    head_dim = D // num_heads
    scale = head_dim ** (-0.5)
    BB = 2

    wq = w_qkv_t.astype(jnp.bfloat16)
    wp = w_proj_t.astype(jnp.bfloat16)
    bq = b_qkv.reshape(1, 3 * D)
    bp = b_proj.reshape(1, D)

    kern = functools.partial(_fused_mha_kernel, num_heads=num_heads,
                             head_dim=head_dim, scale=scale, bb=BB, n=N)
    return pl.pallas_call(
        kern,
        out_shape=jax.ShapeDtypeStruct((B, N, D), x.dtype),
        grid=(B // BB,),
        in_specs=[
            pl.BlockSpec((BB, N, D), lambda b: (b, 0, 0)),
            pl.BlockSpec((D, 3 * D), lambda b: (0, 0)),
            pl.BlockSpec((1, 3 * D), lambda b: (0, 0)),
            pl.BlockSpec((D, D), lambda b: (0, 0)),
            pl.BlockSpec((1, D), lambda b: (0, 0)),
        ],
        out_specs=pl.BlockSpec((BB, N, D), lambda b: (b, 0, 0)),
        compiler_params=pltpu.CompilerParams(
            dimension_semantics=("parallel",),
            vmem_limit_bytes=100 * 1024 * 1024),
    )(x, wq, bq, wp, bp)


# speedup vs baseline: 5.5869x; 1.0461x over previous
import functools

import jax
import jax.numpy as jnp
from jax import lax
from jax.experimental import pallas as pl
from jax.experimental.pallas import tpu as pltpu


def _fused_mha_kernel(x_ref, wqkv_ref, bqkv_ref, wproj_ref, bproj_ref,
                      o_ref, *, num_heads, head_dim, scale, bb, n):
    D = num_heads * head_dim
    xb = x_ref[...].reshape(bb * n, D).astype(jnp.bfloat16)

    qkv = jnp.dot(xb, wqkv_ref[...],
                  preferred_element_type=jnp.float32)
    qkv = qkv + bqkv_ref[...]

    qb = (qkv[:, 0:D] * scale).astype(jnp.bfloat16)
    kb = qkv[:, D:2 * D].astype(jnp.bfloat16)
    vb = qkv[:, 2 * D:3 * D]

    outs = []
    for b in range(bb):
        r0 = b * n
        for h in range(num_heads):
            lo = h * head_dim
            hi = lo + head_dim
            qh = qb[r0:r0 + n, lo:hi]
            kh = kb[r0:r0 + n, lo:hi]
            vh = vb[r0:r0 + n, lo:hi]
            s = lax.dot_general(qh, kh, (((1,), (1,)), ((), ())),
                                preferred_element_type=jnp.float32)
            s = s - jnp.max(s, axis=-1, keepdims=True)
            p = jnp.exp(s)
            l = jnp.sum(p, axis=-1, keepdims=True)
            o = jnp.dot(p, vh,
                        preferred_element_type=jnp.float32)
            outs.append(o * (1.0 / l))

    y = jnp.concatenate(
        [jnp.concatenate(outs[b * num_heads:(b + 1) * num_heads], axis=-1)
         for b in range(bb)], axis=0).astype(jnp.bfloat16)
    out = jnp.dot(y, wproj_ref[...],
                  preferred_element_type=jnp.float32) + bproj_ref[...]
    o_ref[...] = out.reshape(bb, n, D).astype(o_ref.dtype)


def kernel(x, w_qkv_t, w_proj_t, b_qkv, b_proj):
    B, N, D = x.shape
    num_heads = 12
    head_dim = D // num_heads
    scale = head_dim ** (-0.5)
    BB = 2

    wq = w_qkv_t.astype(jnp.bfloat16)
    wp = w_proj_t.astype(jnp.bfloat16)
    bq = b_qkv.reshape(1, 3 * D)
    bp = b_proj.reshape(1, D)

    kern = functools.partial(_fused_mha_kernel, num_heads=num_heads,
                             head_dim=head_dim, scale=scale, bb=BB, n=N)
    return pl.pallas_call(
        kern,
        out_shape=jax.ShapeDtypeStruct((B, N, D), x.dtype),
        grid=(B // BB,),
        in_specs=[
            pl.BlockSpec((BB, N, D), lambda b: (b, 0, 0)),
            pl.BlockSpec((D, 3 * D), lambda b: (0, 0)),
            pl.BlockSpec((1, 3 * D), lambda b: (0, 0)),
            pl.BlockSpec((D, D), lambda b: (0, 0)),
            pl.BlockSpec((1, D), lambda b: (0, 0)),
        ],
        out_specs=pl.BlockSpec((BB, N, D), lambda b: (b, 0, 0)),
        compiler_params=pltpu.CompilerParams(
            dimension_semantics=("parallel",),
            vmem_limit_bytes=100 * 1024 * 1024),
    )(x, wq, bq, wp, bp)


# speedup vs baseline: 5.8783x; 1.0522x over previous
import functools

import jax
import jax.numpy as jnp
from jax import lax
from jax.experimental import pallas as pl
from jax.experimental.pallas import tpu as pltpu


def _fused_mha_kernel(x_ref, wqkv_ref, bqkv_ref, wproj_ref, bproj_ref,
                      o_ref, *, num_heads, head_dim, scale, bb, n):
    D = num_heads * head_dim
    xb = x_ref[...].reshape(bb * n, D).astype(jnp.bfloat16)

    qkv = jnp.dot(xb, wqkv_ref[...],
                  preferred_element_type=jnp.float32)
    qkv = qkv + bqkv_ref[...]

    qb = (qkv[:, 0:D] * scale).astype(jnp.bfloat16)
    kb = qkv[:, D:2 * D].astype(jnp.bfloat16)
    vb = qkv[:, 2 * D:3 * D]

    outs = []
    for b in range(bb):
        r0 = b * n
        for h in range(num_heads):
            lo = h * head_dim
            hi = lo + head_dim
            qh = qb[r0:r0 + n, lo:hi]
            kh = kb[r0:r0 + n, lo:hi]
            vh = vb[r0:r0 + n, lo:hi]
            s = lax.dot_general(qh, kh, (((1,), (1,)), ((), ())),
                                preferred_element_type=jnp.float32)
            s = s - jnp.max(s, axis=-1, keepdims=True)
            p = jnp.exp(s)
            l = jnp.sum(p, axis=-1, keepdims=True)
            o = jnp.dot(p, vh,
                        preferred_element_type=jnp.float32)
            outs.append(o * (1.0 / l))

    y = jnp.concatenate(
        [jnp.concatenate(outs[b * num_heads:(b + 1) * num_heads], axis=-1)
         for b in range(bb)], axis=0).astype(jnp.bfloat16)
    out = jnp.dot(y, wproj_ref[...],
                  preferred_element_type=jnp.float32) + bproj_ref[...]
    o_ref[...] = out.reshape(bb, n, D).astype(o_ref.dtype)


def kernel(x, w_qkv_t, w_proj_t, b_qkv, b_proj):
    B, N, D = x.shape
    num_heads = 12
    head_dim = D // num_heads
    scale = head_dim ** (-0.5)
    BB = 4

    wq = w_qkv_t.astype(jnp.bfloat16)
    wp = w_proj_t.astype(jnp.bfloat16)
    bq = b_qkv.reshape(1, 3 * D)
    bp = b_proj.reshape(1, D)

    kern = functools.partial(_fused_mha_kernel, num_heads=num_heads,
                             head_dim=head_dim, scale=scale, bb=BB, n=N)
    return pl.pallas_call(
        kern,
        out_shape=jax.ShapeDtypeStruct((B, N, D), x.dtype),
        grid=(B // BB,),
        in_specs=[
            pl.BlockSpec((BB, N, D), lambda b: (b, 0, 0)),
            pl.BlockSpec((D, 3 * D), lambda b: (0, 0)),
            pl.BlockSpec((1, 3 * D), lambda b: (0, 0)),
            pl.BlockSpec((D, D), lambda b: (0, 0)),
            pl.BlockSpec((1, D), lambda b: (0, 0)),
        ],
        out_specs=pl.BlockSpec((BB, N, D), lambda b: (b, 0, 0)),
        compiler_params=pltpu.CompilerParams(
            dimension_semantics=("parallel",),
            vmem_limit_bytes=100 * 1024 * 1024),
    )(x, wq, bq, wp, bp)


# speedup vs baseline: 5.9360x; 1.0098x over previous
import functools

import jax
import jax.numpy as jnp
from jax import lax
from jax.experimental import pallas as pl
from jax.experimental.pallas import tpu as pltpu


def _fused_mha_kernel(x_ref, wqkv_ref, bqkv_ref, wproj_ref, bproj_ref,
                      o_ref, *, num_heads, head_dim, scale, bb, n):
    D = num_heads * head_dim
    xb = x_ref[...].reshape(bb * n, D).astype(jnp.bfloat16)

    qkv = jnp.dot(xb, wqkv_ref[...],
                  preferred_element_type=jnp.float32)
    qkv = qkv + bqkv_ref[...]

    qb = (qkv[:, 0:D] * (scale * 1.4426950408889634)).astype(jnp.bfloat16)
    kb = qkv[:, D:2 * D].astype(jnp.bfloat16)
    vb = qkv[:, 2 * D:3 * D]

    outs = []
    for b in range(bb):
        r0 = b * n
        for h in range(num_heads):
            lo = h * head_dim
            hi = lo + head_dim
            qh = qb[r0:r0 + n, lo:hi]
            kh = kb[r0:r0 + n, lo:hi]
            vh = vb[r0:r0 + n, lo:hi]
            s = lax.dot_general(qh, kh, (((1,), (1,)), ((), ())),
                                preferred_element_type=jnp.float32)
            s = s - jnp.max(s, axis=-1, keepdims=True)
            p = jnp.exp2(s)
            l = jnp.sum(p, axis=-1, keepdims=True)
            o = jnp.dot(p, vh,
                        preferred_element_type=jnp.float32)
            outs.append(o * (1.0 / l))

    y = jnp.concatenate(
        [jnp.concatenate(outs[b * num_heads:(b + 1) * num_heads], axis=-1)
         for b in range(bb)], axis=0).astype(jnp.bfloat16)
    out = jnp.dot(y, wproj_ref[...],
                  preferred_element_type=jnp.float32) + bproj_ref[...]
    o_ref[...] = out.reshape(bb, n, D).astype(o_ref.dtype)


def kernel(x, w_qkv_t, w_proj_t, b_qkv, b_proj):
    B, N, D = x.shape
    num_heads = 12
    head_dim = D // num_heads
    scale = head_dim ** (-0.5)
    BB = 4

    wq = w_qkv_t.astype(jnp.bfloat16)
    wp = w_proj_t.astype(jnp.bfloat16)
    bq = b_qkv.reshape(1, 3 * D)
    bp = b_proj.reshape(1, D)

    kern = functools.partial(_fused_mha_kernel, num_heads=num_heads,
                             head_dim=head_dim, scale=scale, bb=BB, n=N)
    return pl.pallas_call(
        kern,
        out_shape=jax.ShapeDtypeStruct((B, N, D), x.dtype),
        grid=(B // BB,),
        in_specs=[
            pl.BlockSpec((BB, N, D), lambda b: (b, 0, 0)),
            pl.BlockSpec((D, 3 * D), lambda b: (0, 0)),
            pl.BlockSpec((1, 3 * D), lambda b: (0, 0)),
            pl.BlockSpec((D, D), lambda b: (0, 0)),
            pl.BlockSpec((1, D), lambda b: (0, 0)),
        ],
        out_specs=pl.BlockSpec((BB, N, D), lambda b: (b, 0, 0)),
        compiler_params=pltpu.CompilerParams(
            dimension_semantics=("parallel",),
            vmem_limit_bytes=100 * 1024 * 1024),
    )(x, wq, bq, wp, bp)


# speedup vs baseline: 6.4132x; 1.0804x over previous
import functools

import jax
import jax.numpy as jnp
from jax import lax
from jax.experimental import pallas as pl
from jax.experimental.pallas import tpu as pltpu


def _fused_mha_kernel(x_ref, wqkv_ref, bqkv_ref, wproj_ref, bproj_ref,
                      o_ref, *, num_heads, head_dim, scale, bb, n):
    D = num_heads * head_dim
    xb = x_ref[...].reshape(bb * n, D).astype(jnp.bfloat16)

    qkv = jnp.dot(xb, wqkv_ref[...],
                  preferred_element_type=jnp.float32)
    qkv = qkv + bqkv_ref[...]

    qb = (qkv[:, 0:D] * (scale * 1.4426950408889634)).astype(jnp.bfloat16)
    kb = qkv[:, D:2 * D].astype(jnp.bfloat16)
    vb = qkv[:, 2 * D:3 * D]

    RT = 128
    row_tiles = []
    for b in range(bb):
        r0 = b * n
        for rt in range(n // RT):
            q0 = r0 + rt * RT
            tile_outs = []
            for h in range(num_heads):
                lo = h * head_dim
                hi = lo + head_dim
                qh = qb[q0:q0 + RT, lo:hi]
                kh = kb[r0:r0 + n, lo:hi]
                vh = vb[r0:r0 + n, lo:hi]
                s = lax.dot_general(qh, kh, (((1,), (1,)), ((), ())),
                                    preferred_element_type=jnp.float32)
                s = s - jnp.max(s, axis=-1, keepdims=True)
                p = jnp.exp2(s)
                l = jnp.sum(p, axis=-1, keepdims=True)
                o = jnp.dot(p, vh,
                            preferred_element_type=jnp.float32)
                tile_outs.append(o * (1.0 / l))
            row_tiles.append(jnp.concatenate(tile_outs, axis=-1))

    y = jnp.concatenate(row_tiles, axis=0).astype(jnp.bfloat16)
    out = jnp.dot(y, wproj_ref[...],
                  preferred_element_type=jnp.float32) + bproj_ref[...]
    o_ref[...] = out.reshape(bb, n, D).astype(o_ref.dtype)


def kernel(x, w_qkv_t, w_proj_t, b_qkv, b_proj):
    B, N, D = x.shape
    num_heads = 12
    head_dim = D // num_heads
    scale = head_dim ** (-0.5)
    BB = 4

    wq = w_qkv_t.astype(jnp.bfloat16)
    wp = w_proj_t.astype(jnp.bfloat16)
    bq = b_qkv.reshape(1, 3 * D)
    bp = b_proj.reshape(1, D)

    kern = functools.partial(_fused_mha_kernel, num_heads=num_heads,
                             head_dim=head_dim, scale=scale, bb=BB, n=N)
    return pl.pallas_call(
        kern,
        out_shape=jax.ShapeDtypeStruct((B, N, D), x.dtype),
        grid=(B // BB,),
        in_specs=[
            pl.BlockSpec((BB, N, D), lambda b: (b, 0, 0)),
            pl.BlockSpec((D, 3 * D), lambda b: (0, 0)),
            pl.BlockSpec((1, 3 * D), lambda b: (0, 0)),
            pl.BlockSpec((D, D), lambda b: (0, 0)),
            pl.BlockSpec((1, D), lambda b: (0, 0)),
        ],
        out_specs=pl.BlockSpec((BB, N, D), lambda b: (b, 0, 0)),
        compiler_params=pltpu.CompilerParams(
            dimension_semantics=("parallel",),
            allow_input_fusion=[False, True, False, True, False],
            vmem_limit_bytes=100 * 1024 * 1024),
    )(x, wq, bq, wp, bp)
